# baseline (speedup 1.0000x reference)
import functools

import jax
import jax.numpy as jnp
from jax import lax
from jax.experimental import pallas as pl
from jax.experimental.pallas import tpu as pltpu
from jax.experimental.pallas import tpu_sc as plsc

N_NODES = 10000
N_EDGES = 320000
D = 128

NC = 2
NS = 16
CHUNK = 128
CPT = 78
TAIL = 16
EPT = CPT * CHUNK + TAIL
PH0 = 40 * CHUNK
PH1_OFF = EPT - PH0
SKIP = PH0 - PH1_OFF
ROWS_PER_TILE = 624
TAIL_ROWS = N_NODES - NS * ROWS_PER_TILE
N_AGG = 10000
NB = 2
NBLK = CPT // NB


def _linear_body(x_ref, w_ref, b_ref, o_ref):
    o_ref[...] = lax.dot_general(
        x_ref[...], w_ref[...], (((1,), (1,)), ((), ())),
        preferred_element_type=jnp.float32,
    ) + b_ref[...]


def _combine_body(a0_ref, a1_ref, h_ref, w_ref, b_ref, o_ref):
    agg = a0_ref[0] + a1_ref[0] - h_ref[...]
    o_ref[...] = lax.dot_general(
        jnp.maximum(agg, 0.0), w_ref[...], (((1,), (1,)), ((), ())),
        preferred_element_type=jnp.float32,
    ) + b_ref[...]


_ROW_BLK = 1000


def _tc_linear(x, w, b):
    return pl.pallas_call(
        _linear_body,
        out_shape=jax.ShapeDtypeStruct((N_NODES, D), jnp.float32),
        grid=(N_NODES // _ROW_BLK,),
        in_specs=[
            pl.BlockSpec((_ROW_BLK, D), lambda i: (i, 0)),
            pl.BlockSpec((D, D), lambda i: (0, 0)),
            pl.BlockSpec((1, D), lambda i: (0, 0)),
        ],
        out_specs=pl.BlockSpec((_ROW_BLK, D), lambda i: (i, 0)),
    )(x, w, b.reshape(1, D))


def _tc_combine(aggs, h, w, b):
    return pl.pallas_call(
        _combine_body,
        out_shape=jax.ShapeDtypeStruct((N_NODES, D), jnp.float32),
        grid=(N_NODES // _ROW_BLK,),
        in_specs=[
            pl.BlockSpec((1, _ROW_BLK, D), lambda i: (0, i, 0)),
            pl.BlockSpec((1, _ROW_BLK, D), lambda i: (1, i, 0)),
            pl.BlockSpec((_ROW_BLK, D), lambda i: (i, 0)),
            pl.BlockSpec((D, D), lambda i: (0, 0)),
            pl.BlockSpec((1, D), lambda i: (0, 0)),
        ],
        out_specs=pl.BlockSpec((_ROW_BLK, D), lambda i: (i, 0)),
    )(aggs, aggs, h, w, b.reshape(1, D))


def _sc_agg_body(h_hbm, ei_hbm, out_hbm, src_v, dst_v,
                 r0, r1, rt, agg_sh, g0, g1, s0, s1):
    rows = [r0, r1]
    gsems = [g0, g1]
    ssems = [s0, s1]
    c = lax.axis_index("c")
    s = lax.axis_index("s")
    wid = c * NS + s
    def _stage(off, size):
        pltpu.sync_copy(ei_hbm.at[pl.ds(wid * EPT + off, size)], src_v)
        pltpu.sync_copy(ei_hbm.at[pl.ds(N_EDGES + wid * EPT + off, size)],
                        dst_v)

    _stage(0, PH0)
    pltpu.sync_copy(h_hbm.at[pl.ds(s * ROWS_PER_TILE, ROWS_PER_TILE)],
                    agg_sh.at[pl.ds(s * ROWS_PER_TILE, ROWS_PER_TILE)])

    @pl.when(s == NS - 1)
    def _init_tail():
        pltpu.sync_copy(h_hbm.at[pl.ds(NS * ROWS_PER_TILE, TAIL_ROWS)],
                        agg_sh.at[pl.ds(NS * ROWS_PER_TILE, TAIL_ROWS)])

    plsc.subcore_barrier()

    def _run(npairs, off):
        for b in range(2):
            pltpu.async_copy(h_hbm.at[src_v.at[pl.ds(off + b * CHUNK, CHUNK)]],
                             rows[b], gsems[b])

        def blk(i, carry):
            base = off + i * 2 * CHUNK
            for b in range(2):
                pltpu.make_async_copy(
                    h_hbm.at[pl.ds(0, CHUNK)], rows[b], gsems[b]).wait()
                pltpu.async_copy(
                    rows[b], agg_sh.at[dst_v.at[pl.ds(base + b * CHUNK, CHUNK)]],
                    ssems[b], add=True)

            @pl.when(i < npairs - 1)
            def _prefetch():
                for b in range(2):
                    pltpu.make_async_copy(
                        h_hbm.at[pl.ds(0, CHUNK)], rows[b], ssems[b]).wait()
                    pltpu.async_copy(
                        h_hbm.at[src_v.at[pl.ds(base + (2 + b) * CHUNK, CHUNK)]],
                        rows[b], gsems[b])

            return carry

        lax.fori_loop(0, npairs, blk, 0)
        for b in range(2):
            pltpu.make_async_copy(
                h_hbm.at[pl.ds(0, CHUNK)], rows[b], ssems[b]).wait()

    _run(PH0 // (2 * CHUNK), 0)
    _stage(PH1_OFF, PH0)
    _run((PH0 - SKIP - TAIL) // (2 * CHUNK), SKIP)
    pltpu.async_copy(h_hbm.at[src_v.at[pl.ds(PH0 - TAIL, TAIL)]], rt,
                     gsems[0]).wait()
    pltpu.async_copy(rt, agg_sh.at[dst_v.at[pl.ds(PH0 - TAIL, TAIL)]],
                     ssems[0], add=True).wait()
    plsc.subcore_barrier()
    pltpu.sync_copy(agg_sh.at[pl.ds(s * ROWS_PER_TILE, ROWS_PER_TILE)],
                    out_hbm.at[c, pl.ds(s * ROWS_PER_TILE, ROWS_PER_TILE)])

    @pl.when(s == NS - 1)
    def _out_tail():
        pltpu.sync_copy(agg_sh.at[pl.ds(NS * ROWS_PER_TILE, TAIL_ROWS)],
                        out_hbm.at[c, pl.ds(NS * ROWS_PER_TILE, TAIL_ROWS)])


_sc_agg = functools.partial(
    pl.kernel,
    out_type=jax.ShapeDtypeStruct((NC, N_NODES, D), jnp.float32),
    mesh=plsc.VectorSubcoreMesh(core_axis_name="c", subcore_axis_name="s",
                                num_cores=NC, num_subcores=NS),
    scratch_types=[
        pltpu.VMEM((PH0,), jnp.int32),
        pltpu.VMEM((PH0,), jnp.int32),
        pltpu.VMEM((CHUNK, D), jnp.float32),
        pltpu.VMEM((CHUNK, D), jnp.float32),
        pltpu.VMEM((TAIL, D), jnp.float32),
        pltpu.VMEM_SHARED((N_AGG, D), jnp.float32),
        pltpu.SemaphoreType.DMA,
        pltpu.SemaphoreType.DMA,
        pltpu.SemaphoreType.DMA,
        pltpu.SemaphoreType.DMA,
    ],
)(_sc_agg_body)


def kernel(x, edge_index, lin_w, lin_b, fc_w, fc_b):
    ei_flat = edge_index.astype(jnp.int32).reshape(-1)

    h = _tc_linear(x, lin_w, lin_b)
    aggs = _sc_agg(h, ei_flat)
    return _tc_combine(aggs, h, fc_w, fc_b)
